# Initial kernel scaffold; baseline (speedup 1.0000x reference)
import jax
import jax.numpy as jnp
from jax.experimental import pallas as pl


def kernel(xc_off_grid, yc_off_grid, xc_on_grid, yc_on_grid, xt):
    raise NotImplementedError("write your pallas kernel here")


# speedup vs baseline: 16.3819x; 16.3819x over previous
import functools

import jax
import jax.numpy as jnp
from jax.experimental import pallas as pl


def _body(x0, x1, x2, x3, y0, y1, y2, y3, xo, yo, xtr, out):
    f32 = jnp.float32
    gxy = (x0[0] + x1[0] + x2[0] + x3[0]) * 0.25
    gx = gxy[:, 0:1]
    gy = gxy[:, 1:2]
    gv = (y0[0] + y1[0] + y2[0] + y3[0]) * 0.25

    S = gxy.shape[0]

    px = xo[0][0:1, :]
    py = xo[0][1:2, :]
    dist = jnp.abs(gx - px) + jnp.abs(gy - py)
    idx = jnp.argmin(dist, axis=0, keepdims=True)
    row = jax.lax.broadcasted_iota(jnp.int32, dist.shape, 0)
    onehot = (idx == row).astype(f32)

    sums = jax.lax.dot_general(
        onehot, yo[0], (((1,), (0,)), ((), ())),
        preferred_element_type=f32, precision=jax.lax.Precision.HIGHEST)
    counts = jnp.sum(onehot, axis=1, keepdims=True)
    avg = (gv + sums) / (counts + 1.0)

    tx = xtr[0][0:1, :]
    ty = xtr[0][1:2, :]
    dist_t = jnp.abs(gx - tx) + jnp.abs(gy - ty)
    idx_t = jnp.argmin(dist_t, axis=0, keepdims=True)
    row_t = jax.lax.broadcasted_iota(jnp.int32, dist_t.shape, 0)
    onehot_t = (idx_t == row_t).astype(f32)
    out[0] = jax.lax.dot_general(
        onehot_t, avg, (((0,), (0,)), ((), ())),
        preferred_element_type=f32, precision=jax.lax.Precision.HIGHEST)


@jax.jit
def kernel(xc_off_grid, yc_off_grid, xc_on_grid, yc_on_grid, xt):
    B, U, Yd = yc_off_grid.shape
    _, H, W, _ = xc_on_grid.shape
    Hc, Wc = H // 2, W // 2
    S = Hc * Wc
    T = xt.shape[1]

    xviews = [xc_on_grid[:, i::2, j::2, :].reshape(B, S, 2)
              for i in range(2) for j in range(2)]
    yviews = [yc_on_grid[:, i::2, j::2, :].reshape(B, S, Yd)
              for i in range(2) for j in range(2)]
    xoff_t = xc_off_grid.transpose(0, 2, 1)
    xt_t = xt.transpose(0, 2, 1)

    def spec(shape):
        return pl.BlockSpec((1,) + shape, lambda b: (b, 0, 0))

    return pl.pallas_call(
        _body,
        grid=(B,),
        in_specs=[spec((S, 2))] * 4 + [spec((S, Yd))] * 4
                 + [spec((2, U)), spec((U, Yd)), spec((2, T))],
        out_specs=spec((T, Yd)),
        out_shape=jax.ShapeDtypeStruct((B, T, Yd), jnp.float32),
    )(*xviews, *yviews, xoff_t, yc_off_grid, xt_t)


# speedup vs baseline: 17.5204x; 1.0695x over previous
import functools

import jax
import jax.numpy as jnp
from jax import lax
from jax.experimental import pallas as pl
from jax.experimental.pallas import tpu as pltpu
from jax.experimental.pallas import tpu_sc as plsc


def _tc_body(x0, x1, x2, x3, y0, y1, y2, y3, xo, xtr, idx_out, idxt_out,
             gv_out):
    S_ = x0.shape[1]
    gxy = (x0[0] + x1[0] + x2[0] + x3[0]) * 0.25
    gx = gxy[:, 0:1]
    gy = gxy[:, 1:2]
    gv = (y0[0] + y1[0] + y2[0] + y3[0]) * 0.25
    gv_out[0] = jnp.concatenate(
        [gv, jnp.zeros_like(gv)], axis=1)

    px = xo[0][0:1, :]
    py = xo[0][1:2, :]
    dist = jnp.abs(gx - px) + jnp.abs(gy - py)
    idx_out[0] = jnp.argmin(dist, axis=0, keepdims=True)

    tx = xtr[0][0:1, :]
    ty = xtr[0][1:2, :]
    dist_t = jnp.abs(gx - tx) + jnp.abs(gy - ty)
    idxt_out[0] = (jnp.argmin(dist_t, axis=0, keepdims=True)
                   + S_ * pl.program_id(0))


def _sc_kernel(B, S, U, T, Yd2):
    mesh = plsc.VectorSubcoreMesh(core_axis_name="c", subcore_axis_name="s")
    NS = 16
    BPC = B // 2
    UP = U // NS
    TP = T // NS
    SP = S // NS

    W = 128

    @functools.partial(
        pl.kernel, mesh=mesh,
        out_type=[jax.ShapeDtypeStruct((B, T, W), jnp.float32),
                  jax.ShapeDtypeStruct((B * S, W), jnp.float32)],
        scratch_types=[
            pltpu.VMEM((UP,), jnp.int32),
            pltpu.VMEM((UP, W), jnp.float32),
            pltpu.VMEM((SP, W), jnp.float32),
            pltpu.VMEM((SP, Yd2), jnp.float32),
            pltpu.VMEM((SP, W), jnp.float32),
            pltpu.VMEM((TP,), jnp.int32),
            pltpu.VMEM((TP, W), jnp.float32),
            pltpu.VMEM_SHARED((S, W), jnp.float32),
            pltpu.SemaphoreType.DMA,
        ],
    )
    def k(idx_hbm, yaug_hbm, gv_hbm, idxt_hbm, zeros_hbm, out_hbm, avg_hbm,
          idx_v, rows_v, acc_v, gv_v, avg_v, idxt_v, trows_v,
          acc_sh, sem):
        core = lax.axis_index("c")
        sid = lax.axis_index("s")
        for lb in range(BPC):
            b = core * BPC + lb
            pltpu.sync_copy(zeros_hbm.at[pl.ds(sid * SP, SP)],
                            acc_sh.at[pl.ds(sid * SP, SP)])
            pltpu.sync_copy(idx_hbm.at[b, pl.ds(sid * UP, UP)], idx_v)
            pltpu.sync_copy(yaug_hbm.at[b, pl.ds(sid * UP, UP)], rows_v)
            plsc.subcore_barrier()
            pltpu.sync_copy(rows_v, acc_sh.at[idx_v], add=True)
            plsc.subcore_barrier()
            pltpu.sync_copy(acc_sh.at[pl.ds(sid * SP, SP)], acc_v)
            pltpu.sync_copy(gv_hbm.at[b, pl.ds(sid * SP, SP)], gv_v)
            for r in range(SP):
                lo = acc_v[r, 0:Yd2]
                hi = acc_v[r, Yd2:2 * Yd2]
                avg_v[r, 0:Yd2] = (gv_v[r, :] + lo) / (hi + 1.0)
            pltpu.sync_copy(avg_v, avg_hbm.at[pl.ds(b * S + sid * SP, SP)])
            plsc.subcore_barrier()
            pltpu.sync_copy(idxt_hbm.at[b, pl.ds(sid * TP, TP)], idxt_v)
            pltpu.async_copy(avg_hbm.at[idxt_v], trows_v, sem).wait()
            pltpu.sync_copy(trows_v, out_hbm.at[b, pl.ds(sid * TP, TP)])
            plsc.subcore_barrier()

    return k


@jax.jit
def kernel(xc_off_grid, yc_off_grid, xc_on_grid, yc_on_grid, xt):
    B, U, Yd = yc_off_grid.shape
    _, H, W, _ = xc_on_grid.shape
    S = (H // 2) * (W // 2)
    T = xt.shape[1]
    Yd2 = 2 * Yd

    xviews = [xc_on_grid[:, i::2, j::2, :].reshape(B, S, 2)
              for i in range(2) for j in range(2)]
    yviews = [yc_on_grid[:, i::2, j::2, :].reshape(B, S, Yd)
              for i in range(2) for j in range(2)]
    xoff_t = xc_off_grid.transpose(0, 2, 1)
    xt_t = xt.transpose(0, 2, 1)

    def spec(shape):
        return pl.BlockSpec((1,) + shape, lambda b: (b, 0, 0))

    idx, idxt, gv16 = pl.pallas_call(
        _tc_body,
        grid=(B,),
        in_specs=[spec((S, 2))] * 4 + [spec((S, Yd))] * 4
                 + [spec((2, U)), spec((2, T))],
        out_specs=[spec((1, U)), spec((1, T)), spec((S, Yd2))],
        out_shape=[
            jax.ShapeDtypeStruct((B, 1, U), jnp.int32),
            jax.ShapeDtypeStruct((B, 1, T), jnp.int32),
            jax.ShapeDtypeStruct((B, S, Yd2), jnp.float32),
        ],
    )(*xviews, *yviews, xoff_t, xt_t)

    idx = idx.reshape(B, U)
    idxt = idxt.reshape(B, T)
    yaug = jnp.concatenate(
        [yc_off_grid,
         jnp.zeros((B, U, Yd), jnp.float32),
         jnp.ones((B, U, Yd2), jnp.float32),
         jnp.zeros((B, U, 128 - 2 * Yd2), jnp.float32)], axis=-1)
    zeros_tab = jnp.zeros((S, 128), jnp.float32)

    out_w, avg_w = _sc_kernel(B, S, U, T, Yd2)(idx, yaug, gv16, idxt,
                                               zeros_tab)
    del avg_w
    return out_w[:, :, :Yd]


# speedup vs baseline: 64.2019x; 3.6644x over previous
import functools

import jax
import jax.numpy as jnp
from jax import lax
from jax.experimental import pallas as pl
from jax.experimental.pallas import tpu as pltpu
from jax.experimental.pallas import tpu_sc as plsc


def _sel(shape, fn):
    i = jax.lax.broadcasted_iota(jnp.int32, shape, 0)
    j = jax.lax.broadcasted_iota(jnp.int32, shape, 1)
    return fn(i, j).astype(jnp.float32)


def _dot(a, b):
    return jax.lax.dot_general(a, b, (((1,), (0,)), ((), ())),
                               preferred_element_type=jnp.float32,
                               precision=jax.lax.Precision.HIGHEST)


def _coarsen_body(xg, yg, gxy_out, gv_out):
    x = xg[0]
    y = yg[0]
    H = x.shape[0]
    Hc = H // 2
    Yd = y.shape[1] // x.shape[1] * 2

    R = _sel((Hc, H), lambda i, j: (j >> 1) == i)
    xr = _dot(R, x)
    yr = _dot(R, y)
    P = _sel((x.shape[1], x.shape[1] // 2),
             lambda l, m: m == ((l >> 2) * 2 + (l & 1)))
    gxy_out[0] = _dot(xr, P) * 0.25
    qs = (2 * Yd).bit_length() - 1
    Q2 = _sel((y.shape[1], y.shape[1]),
              lambda l, m: ((m >> qs) == (l >> qs))
              & ((m & (2 * Yd - 1)) == (l & (Yd - 1))))
    gv_out[0] = _dot(yr, Q2) * 0.25


def _tc_body(gxy_ref, xo, xtr, idx_out, idxt_out):
    gxy = gxy_ref[0]
    S_ = gxy.shape[0]
    gx = gxy[:, 0:1]
    gy = gxy[:, 1:2]

    px = xo[0][0:1, :]
    py = xo[0][1:2, :]
    dist = jnp.abs(gx - px) + jnp.abs(gy - py)
    idx_out[0] = jnp.argmin(dist, axis=0, keepdims=True)

    tx = xtr[0][0:1, :]
    ty = xtr[0][1:2, :]
    dist_t = jnp.abs(gx - tx) + jnp.abs(gy - ty)
    idxt_out[0] = (jnp.argmin(dist_t, axis=0, keepdims=True)
                   + S_ * pl.program_id(0))


def _sc_kernel(B, S, U, T, Yd2):
    mesh = plsc.VectorSubcoreMesh(core_axis_name="c", subcore_axis_name="s")
    NS = 16
    BPC = B // 2
    UP = U // NS
    TP = T // NS
    SP = S // NS

    W = 128

    @functools.partial(
        pl.kernel, mesh=mesh,
        out_type=[jax.ShapeDtypeStruct((B, T, W), jnp.float32),
                  jax.ShapeDtypeStruct((B * S, W), jnp.float32)],
        scratch_types=[
            pltpu.VMEM((UP,), jnp.int32),
            pltpu.VMEM((UP, W), jnp.float32),
            pltpu.VMEM((SP, W), jnp.float32),
            pltpu.VMEM((SP, Yd2), jnp.float32),
            pltpu.VMEM((SP, W), jnp.float32),
            pltpu.VMEM((TP,), jnp.int32),
            pltpu.VMEM((TP, W), jnp.float32),
            pltpu.VMEM_SHARED((S, W), jnp.float32),
            pltpu.SemaphoreType.DMA,
        ],
    )
    def k(idx_hbm, yaug_hbm, gv_hbm, idxt_hbm, zeros_hbm, out_hbm, avg_hbm,
          idx_v, rows_v, acc_v, gv_v, avg_v, idxt_v, trows_v,
          acc_sh, sem):
        core = lax.axis_index("c")
        sid = lax.axis_index("s")
        for lb in range(BPC):
            b = core * BPC + lb
            pltpu.sync_copy(zeros_hbm.at[pl.ds(sid * SP, SP)],
                            acc_sh.at[pl.ds(sid * SP, SP)])
            pltpu.sync_copy(idx_hbm.at[b, pl.ds(sid * UP, UP)], idx_v)
            pltpu.sync_copy(yaug_hbm.at[b, pl.ds(sid * UP, UP)], rows_v)
            plsc.subcore_barrier()
            pltpu.sync_copy(rows_v, acc_sh.at[idx_v], add=True)
            plsc.subcore_barrier()
            pltpu.sync_copy(acc_sh.at[pl.ds(sid * SP, SP)], acc_v)
            pltpu.sync_copy(gv_hbm.at[b, pl.ds(sid * SP, SP)], gv_v)
            for r in range(SP):
                lo = acc_v[r, 0:Yd2]
                hi = acc_v[r, Yd2:2 * Yd2]
                avg_v[r, 0:Yd2] = (gv_v[r, :] + lo) / (hi + 1.0)
            pltpu.sync_copy(avg_v, avg_hbm.at[pl.ds(b * S + sid * SP, SP)])
            plsc.subcore_barrier()
            pltpu.sync_copy(idxt_hbm.at[b, pl.ds(sid * TP, TP)], idxt_v)
            pltpu.async_copy(avg_hbm.at[idxt_v], trows_v, sem).wait()
            pltpu.sync_copy(trows_v, out_hbm.at[b, pl.ds(sid * TP, TP)])
            plsc.subcore_barrier()

    return k


@jax.jit
def kernel(xc_off_grid, yc_off_grid, xc_on_grid, yc_on_grid, xt):
    B, U, Yd = yc_off_grid.shape
    _, H, W, _ = xc_on_grid.shape
    S = (H // 2) * (W // 2)
    T = xt.shape[1]
    Yd2 = 2 * Yd

    xg = xc_on_grid.reshape(B, H, W * 2)
    yg = yc_on_grid.reshape(B, H, W * Yd)
    xoff_t = xc_off_grid.transpose(0, 2, 1)
    xt_t = xt.transpose(0, 2, 1)
    Hc, Wc = H // 2, W // 2

    def spec(shape):
        return pl.BlockSpec((1,) + shape, lambda b: (b, 0, 0))

    gxy_p, gv_p = pl.pallas_call(
        _coarsen_body,
        grid=(B,),
        in_specs=[spec((H, W * 2)), spec((H, W * Yd))],
        out_specs=[spec((Hc, Wc * 2)), spec((Hc, Wc * Yd2))],
        out_shape=[
            jax.ShapeDtypeStruct((B, Hc, Wc * 2), jnp.float32),
            jax.ShapeDtypeStruct((B, Hc, Wc * Yd2), jnp.float32),
        ],
    )(xg, yg)
    gxy = gxy_p.reshape(B, S, 2)
    gv16 = gv_p.reshape(B, S, Yd2)

    idx, idxt = pl.pallas_call(
        _tc_body,
        grid=(B,),
        in_specs=[spec((S, 2)), spec((2, U)), spec((2, T))],
        out_specs=[spec((1, U)), spec((1, T))],
        out_shape=[
            jax.ShapeDtypeStruct((B, 1, U), jnp.int32),
            jax.ShapeDtypeStruct((B, 1, T), jnp.int32),
        ],
    )(gxy, xoff_t, xt_t)

    idx = idx.reshape(B, U)
    idxt = idxt.reshape(B, T)
    yaug = jnp.concatenate(
        [yc_off_grid,
         jnp.zeros((B, U, Yd), jnp.float32),
         jnp.ones((B, U, Yd2), jnp.float32),
         jnp.zeros((B, U, 128 - 2 * Yd2), jnp.float32)], axis=-1)
    zeros_tab = jnp.zeros((S, 128), jnp.float32)

    out_w, avg_w = _sc_kernel(B, S, U, T, Yd2)(idx, yaug, gv16, idxt,
                                               zeros_tab)
    del avg_w
    return out_w[:, :, :Yd]
